# baseline (speedup 1.0000x reference)
import functools

import jax
import jax.numpy as jnp
import numpy as np
from jax import lax
from jax.experimental import pallas as pl
from jax.experimental.pallas import tpu as pltpu
from jax.experimental.pallas import tpu_sc as plsc

_CUBE_CORNERS = np.array(
    [[0, 0, 0], [1, 0, 0], [0, 1, 0], [1, 1, 0],
     [0, 0, 1], [1, 0, 1], [0, 1, 1], [1, 1, 1]], dtype=np.float32)
_CUBE_EDGES = np.array(
    [0, 1, 1, 5, 4, 5, 0, 4, 2, 3, 3, 7, 6, 7, 2, 6, 2, 0, 3, 1, 7, 5, 6, 4],
    dtype=np.int32).reshape(12, 2)

_NC, _NS, _L = 2, 16, 16
_NW = _NC * _NS
_K = 4096


def _gather_body(k, table_hbm, idx_hbm, out_hbm, idx_v, rows_v, sem):
    wid = lax.axis_index("s") * _NC + lax.axis_index("c")
    n_idx = idx_hbm.shape[0]
    per_w = n_idx // _NW
    base = wid * per_w
    steps = per_w // k

    def step(j, _):
        off = base + j * k
        pltpu.sync_copy(idx_hbm.at[pl.ds(off, k)], idx_v)
        pltpu.async_copy(table_hbm.at[idx_v], rows_v, sem).wait()
        pltpu.sync_copy(rows_v, out_hbm.at[pl.ds(off, k)])
        return 0

    lax.fori_loop(0, steps, step, 0)


def _sc_gather(table, flat_idx, k=_K):
    n_idx = flat_idx.shape[0]
    d = table.shape[1]
    mesh = plsc.VectorSubcoreMesh(core_axis_name="c", subcore_axis_name="s",
                                  num_cores=_NC, num_subcores=_NS)
    kern = pl.kernel(
        functools.partial(_gather_body, k),
        out_type=jax.ShapeDtypeStruct((n_idx, d), jnp.float32),
        mesh=mesh,
        compiler_params=pltpu.CompilerParams(use_tc_tiling_on_sc=False),
        scratch_types=[
            pltpu.VMEM((k,), jnp.int32),
            pltpu.VMEM((k, d), jnp.float32),
            pltpu.SemaphoreType.DMA,
        ],
    )
    return kern(table, flat_idx)


def _dense_body(g_ref, b_ref, a_ref, gm_ref, o_ref):
    ws = 0.99
    gt = g_ref[...].T
    s = [gt[16 * c + 3] for c in range(8)]
    px = [gt[16 * c + 0] for c in range(8)]
    py = [gt[16 * c + 1] for c in range(8)]
    pz = [gt[16 * c + 2] for c in range(8)]

    cnt = s[0] < 0
    cnt = cnt.astype(jnp.int32)
    for c in range(1, 8):
        cnt = cnt + (s[c] < 0).astype(jnp.int32)
    surf = (cnt > 0) & (cnt < 8)

    at = a_ref[...].T
    bt = b_ref[...].T
    a_n = [jnp.tanh(at[c]) * ws + 1.0 for c in range(8)]
    gamma_n = jax.nn.sigmoid(gm_ref[...].T[0]) * ws + (1.0 - ws) / 2.0

    rows = []
    for e in range(12):
        c0, c1 = int(_CUBE_EDGES[e, 0]), int(_CUBE_EDGES[e, 1])
        s0, s1 = s[c0], s[c1]
        w0 = a_n[c0] * jnp.abs(s0)
        w1 = a_n[c1] * jnp.abs(s1)
        t = w0 / (w0 + w1 + 1e-8)
        active = (s0 * s1) < 0
        active = active & surf
        beta_n = jnp.tanh(bt[e]) * ws + 1.0
        scale = jnp.where(active, beta_n * gamma_n, 0.0)
        u = 1.0 - t
        rows.append((px[c0] * u + px[c1] * t) * scale)
        rows.append((py[c0] * u + py[c1] * t) * scale)
        rows.append((pz[c0] * u + pz[c1] * t) * scale)

    o_ref[...] = jnp.stack(rows, axis=0).T


def _tc_dense(g, beta, alpha, gamma2d, n_cubes):
    B = 512
    grid = n_cubes // B

    def spec(cols):
        return pl.BlockSpec((B, cols), lambda i: (i, 0))

    return pl.pallas_call(
        _dense_body,
        out_shape=jax.ShapeDtypeStruct((n_cubes, 36), jnp.float32),
        grid=(grid,),
        in_specs=[spec(128), spec(12), spec(8), spec(1)],
        out_specs=spec(36),
    )(g, beta, alpha, gamma2d)


def kernel(voxel_coords, sdf, cube_idx, resolution, deform, beta, alpha, gamma):
    N = cube_idx.shape[0]
    M = sdf.shape[0]

    flat = cube_idx.reshape(-1)
    iidx = jnp.arange(N * 8, dtype=jnp.int32)
    W = jnp.zeros((M,), jnp.int32).at[flat].max(iidx)
    wn = W >> 3
    wc = W & 7
    vox16 = jnp.pad(voxel_coords.astype(jnp.float32), ((0, 0), (0, 13)))
    chunk = 2048
    mp = -(-M // (_NW * chunk)) * (_NW * chunk)
    wn_p = jnp.pad(wn, (0, mp - M))
    vg = _sc_gather(vox16, wn_p, chunk)[:M]
    corner = jnp.stack(
        [(wc & 1), ((wc >> 1) & 1), ((wc >> 2) & 1)], axis=1
    ).astype(jnp.float32)
    pos = vg[:, :3] + corner
    world_scale = 2.0 / resolution
    world = (pos + 0.5) * world_scale - 1.0 + deform
    table = jnp.concatenate([world, sdf[:, None]], axis=1)
    table16 = jnp.pad(table, ((0, 0), (0, 12)))

    g = _sc_gather(table16, flat)

    out = _tc_dense(g.reshape(N, 128), beta, alpha,
                    gamma.reshape(N, 1), N)
    return out.reshape(N * 12, 3)
